# baseline (speedup 1.0000x reference)
import jax
import jax.numpy as jnp
import numpy as np
from jax.experimental import pallas as pl

_OBS = 8
_PRED = 12
_N = 256
_H = 16

_HK = jax.random.key(1)
_H0T = np.asarray(jax.random.normal(jax.random.fold_in(_HK, 0), (_N, _H),
                                    dtype=np.float32)).T.copy()
_C0T = np.asarray(jax.random.normal(jax.random.fold_in(_HK, 1), (_N, _H),
                                    dtype=np.float32)).T.copy()

_SPEC = [
    ('trajT', 2 * _OBS, _N), ('posT0', 2, _N), ('goalT', 2, _N),
    ('h0T', _H, _N), ('c0T', _H, _N),
    ('enc_W', _H, 2), ('enc_b', _H, 1),
    ('goal_W', _H, 2), ('goal_b', _H, 1),
    ('l1_Wih', 4 * _H, _H), ('l1_Whh', 4 * _H, _H), ('l1_b', 4 * _H, 1),
    ('l2_Wih', 4 * _H, _H), ('l2_Whh', 4 * _H, _H), ('l2_b', 4 * _H, 1),
    ('dec_Wc', _H, _H), ('dec_Wo', _H, 2), ('dec_b', _H, 1),
    ('Wmu', 2, _H), ('bmu', 2, 1), ('Wsc', 2, _H), ('bsc', 2, 1),
    ('corr_W', 2, _H), ('corr_b', 2, 1),
    ('pl_WrT', _H, 2), ('pl_br', _H, 1),
    ('pl_We1T', _H, _H), ('pl_We2T', _H, _H),
    ('pl_be', _H, 1), ('pl_wa', _H, 1), ('pl_ba', 1, 1),
    ('plc_WrT', _H, 2), ('plc_br', _H, 1),
    ('plc_We1T', _H, _H), ('plc_We2T', _H, _H),
    ('plc_be', _H, 1), ('plc_wa', _H, 1), ('plc_ba', 1, 1),
]
_OFF = {}
_rows = 0
for _nm, _rr, _cc in _SPEC:
    _OFF[_nm] = _rows
    _rows += -(-_rr // 8) * 8
_P_ROWS = _rows


def _trajgen_kernel(P_ref, mask_ref, out_ref):
    f32 = jnp.float32
    H = _H
    N = _N

    def ld(nm):
        o = _OFF[nm]
        for s, rr, cc in _SPEC:
            if s == nm:
                return P_ref[o:o + rr, 0:cc]

    def mm(a, b):
        return jnp.dot(a, b, preferred_element_type=f32)

    def lstm(xT, hT, cT, Wih, Whh, b):
        g = mm(Wih, xT) + mm(Whh, hT) + b
        s = jax.nn.sigmoid(g[0:3 * H])
        gg = jnp.tanh(g[3 * H:4 * H])
        c2 = s[H:2 * H] * cT + s[0:H] * gg
        h2 = s[2 * H:3 * H] * jnp.tanh(c2)
        return h2, c2

    def pooling(posT, hT, nei, WrT, br, We1T, We2T, be, wa, ba):
        bf = jnp.bfloat16
        aT = mm(WrT, posT)
        a2 = (aT + br).astype(bf)
        an = aT.astype(bf)
        r = jnp.maximum(a2[:, :, None] - an[:, None, :], bf(0))
        et = jax.lax.dot_general(We1T.astype(bf), r, (((1,), (0,)), ((), ())),
                                 preferred_element_type=f32).astype(bf)
        bT = ((mm(We2T, hT) + be).astype(bf))
        e = jnp.maximum(et + bT[:, None, :], bf(0))
        logits = jnp.sum(wa.astype(bf)[:, :, None] * e, axis=0).astype(f32) + ba
        msk = nei > 0
        lm = jnp.where(msk, logits, jnp.float32(-1e9))
        mx = jnp.max(lm, axis=1, keepdims=True)
        ex = jnp.exp(lm - mx)
        den = jnp.sum(ex, axis=1, keepdims=True)
        attn = jnp.where(msk, ex / den, 0.0).astype(bf)
        return jnp.sum(attn[None, :, :] * e, axis=2).astype(f32)

    enc_W = ld('enc_W')
    enc_b = ld('enc_b')
    l1_Wih = ld('l1_Wih')
    l1_Whh = ld('l1_Whh')
    l1_b = ld('l1_b')
    l2_Wih = ld('l2_Wih')
    l2_Whh = ld('l2_Whh')
    l2_b = ld('l2_b')
    dec_Wc = ld('dec_Wc')
    dec_Wo = ld('dec_Wo')
    dec_b = ld('dec_b')
    Wmu = ld('Wmu')
    bmu = ld('bmu')
    Wsc = ld('Wsc')
    bsc = ld('bsc')
    corr_W = ld('corr_W')
    corr_b = ld('corr_b')
    pl_p = (ld('pl_WrT'), ld('pl_br'), ld('pl_We1T'), ld('pl_We2T'),
            ld('pl_be'), ld('pl_wa'), ld('pl_ba'))
    plc_p = (ld('plc_WrT'), ld('plc_br'), ld('plc_We1T'), ld('plc_We2T'),
             ld('plc_be'), ld('plc_wa'), ld('plc_ba'))

    hT = ld('h0T')
    cT = ld('c0T')
    to = _OFF['trajT']
    for t in range(_OBS):
        xT = jnp.maximum(mm(enc_W, P_ref[to + 2 * t:to + 2 * t + 2, :])
                         + enc_b, 0.0)
        hT, cT = lstm(xT, hT, cT, l1_Wih, l1_Whh, l1_b)

    posT0 = ld('posT0')
    relgT = mm(ld('goal_W'), ld('goalT') - posT0) + ld('goal_b')

    def body(t, carry):
        outT, phT, pcT, posT, ctxT = carry
        xT = jnp.maximum(mm(dec_Wc, ctxT) + mm(dec_Wo, outT) + dec_b, 0.0)
        phT, pcT = lstm(xT, phT, pcT, l2_Wih, l2_Whh, l2_b)
        nei = mask_ref[pl.ds(t, 1)][0]
        ctx1 = pooling(posT, phT, nei, *pl_p)
        concT = ctx1 + phT + relgT
        muT = mm(Wmu, concT) + bmu
        scT = jnp.clip(mm(Wsc, concT) + bsc, -9.0, 4.0)
        pos_s = posT + muT
        ctx2 = pooling(pos_s, phT, nei, *plc_p)
        outP = mm(corr_W, ctx2) + corr_b + muT
        out_ref[pl.ds(t, 1)] = outP[None]
        out_ref[pl.ds(_PRED + t, 1)] = muT[None]
        out_ref[pl.ds(2 * _PRED + t, 1)] = scT[None]
        return (outP, phT, pcT, posT + outP, ctx1)

    init = (P_ref[to + 2 * (_OBS - 1):to + 2 * _OBS, :], hT,
            jnp.zeros_like(hT), posT0, jnp.zeros_like(hT))
    jax.lax.fori_loop(0, _PRED, body, init)


def kernel(traj_rel, obs_traj_pos, pred_traj_gt_pos, seq_start_end,
           nei_index, nei_num_index, sample_goal, params):
    p = params
    f32 = jnp.float32
    col = lambda v: v.reshape(-1, 1).astype(f32)

    def gate_reorder(W):
        return jnp.concatenate([W[:2 * _H], W[3 * _H:], W[2 * _H:3 * _H]], 0)

    vals = {
        'trajT': jnp.transpose(traj_rel[:_OBS], (0, 2, 1)).reshape(2 * _OBS, _N),
        'posT0': obs_traj_pos[-1].T,
        'goalT': sample_goal.T,
        'h0T': jnp.asarray(_H0T), 'c0T': jnp.asarray(_C0T),
        'enc_W': p['enc_W'], 'enc_b': col(p['enc_b']),
        'goal_W': p['goal_W'], 'goal_b': col(p['goal_b']),
        'l1_Wih': gate_reorder(p['lstm1_Wih']),
        'l1_Whh': gate_reorder(p['lstm1_Whh']),
        'l1_b': gate_reorder(col(p['lstm1_bih'] + p['lstm1_bhh'])),
        'l2_Wih': gate_reorder(p['lstm2_Wih']),
        'l2_Whh': gate_reorder(p['lstm2_Whh']),
        'l2_b': gate_reorder(col(p['lstm2_bih'] + p['lstm2_bhh'])),
        'dec_Wc': p['dec_W'][:, :_H], 'dec_Wo': p['dec_W'][:, _H:],
        'dec_b': col(p['dec_b']),
        'Wmu': p['h2p_W'][:2], 'bmu': col(p['h2p_b'][:2]),
        'Wsc': p['h2p_W'][2:], 'bsc': col(p['h2p_b'][2:]),
        'corr_W': p['corr_W'], 'corr_b': col(p['corr_b']),
    }
    for pre in ('pl', 'plc'):
        vals[pre + '_WrT'] = p[pre + '_Wr'].T
        vals[pre + '_br'] = col(p[pre + '_br'])
        vals[pre + '_We1T'] = p[pre + '_We'][:_H].T
        vals[pre + '_We2T'] = p[pre + '_We'][_H:].T
        vals[pre + '_be'] = col(p[pre + '_be'])
        vals[pre + '_wa'] = p[pre + '_wa'].reshape(_H, 1)
        vals[pre + '_ba'] = p[pre + '_ba'].reshape(1, 1)

    blocks = []
    for nm, rr, cc in _SPEC:
        v = vals[nm].astype(f32)
        pr = -(-rr // 8) * 8
        blocks.append(jnp.pad(v, ((0, pr - rr), (0, _N - v.shape[1]))))
    P = jnp.concatenate(blocks, axis=0)

    out = pl.pallas_call(
        _trajgen_kernel,
        out_shape=jax.ShapeDtypeStruct((3 * _PRED, 2, _N), f32),
    )(P, nei_index)
    tr = lambda x: jnp.transpose(x, (0, 2, 1))
    return (tr(out[:_PRED]), tr(out[_PRED:2 * _PRED]), tr(out[2 * _PRED:]))


# speedup vs baseline: 1.0953x; 1.0953x over previous
import jax
import jax.numpy as jnp
import numpy as np
from jax.experimental import pallas as pl

_OBS = 8
_PRED = 12
_N = 256
_H = 16

def _hc_init():
    hk = jax.random.key(1)
    h = jax.random.normal(jax.random.fold_in(hk, 0), (_N, _H),
                          dtype=jnp.float32).T
    c = jax.random.normal(jax.random.fold_in(hk, 1), (_N, _H),
                          dtype=jnp.float32).T
    return h, c

try:
    _H0T, _C0T = (np.asarray(x).copy() for x in _hc_init())
except Exception:
    _H0T = _C0T = None

_SPEC = [
    ('trajT', 2 * _OBS, _N), ('posT0', 2, _N), ('goalT', 2, _N),
    ('h0T', _H, _N), ('c0T', _H, _N),
    ('enc_W', _H, 2), ('enc_b', _H, 1),
    ('goal_W', _H, 2), ('goal_b', _H, 1),
    ('l1_Wih', 4 * _H, _H), ('l1_Whh', 4 * _H, _H), ('l1_b', 4 * _H, 1),
    ('l2_Wih', 4 * _H, _H), ('l2_Whh', 4 * _H, _H), ('l2_b', 4 * _H, 1),
    ('dec_Wc', _H, _H), ('dec_Wo', _H, 2), ('dec_b', _H, 1),
    ('Wmu', 2, _H), ('bmu', 2, 1), ('Wsc', 2, _H), ('bsc', 2, 1),
    ('corr_W', 2, _H), ('corr_b', 2, 1),
    ('pl_WrT', _H, 2), ('pl_br', _H, 1),
    ('pl_We1T', _H, _H), ('pl_We2T', _H, _H),
    ('pl_be', _H, 1), ('pl_wa', _H, 1), ('pl_ba', 1, 1),
    ('plc_WrT', _H, 2), ('plc_br', _H, 1),
    ('plc_We1T', _H, _H), ('plc_We2T', _H, _H),
    ('plc_be', _H, 1), ('plc_wa', _H, 1), ('plc_ba', 1, 1),
]
_OFF = {}
_rows = 0
for _nm, _rr, _cc in _SPEC:
    _OFF[_nm] = _rows
    _rows += -(-_rr // 8) * 8
_P_ROWS = _rows


def _trajgen_kernel(P_ref, mask_ref, out_ref):
    f32 = jnp.float32
    H = _H
    N = _N

    def ld(nm):
        o = _OFF[nm]
        for s, rr, cc in _SPEC:
            if s == nm:
                return P_ref[o:o + rr, 0:cc]

    def mm(a, b):
        return jnp.dot(a, b, preferred_element_type=f32)

    def lstm(xT, hT, cT, Wih, Whh, b):
        g = mm(Wih, xT) + mm(Whh, hT) + b
        s = jax.nn.sigmoid(g[0:3 * H])
        gg = jnp.tanh(g[3 * H:4 * H])
        c2 = s[H:2 * H] * cT + s[0:H] * gg
        h2 = s[2 * H:3 * H] * jnp.tanh(c2)
        return h2, c2

    def pooling(posT, hT, nei, WrT, br, We1T, We2T, be, wa, ba):
        bf = jnp.bfloat16
        aT = mm(WrT, posT)
        a2t = jnp.transpose(aT + br).astype(bf)
        an = aT.astype(bf)
        r3 = jnp.maximum(a2t[:, :, None] - an[None, :, :], bf(0))
        W1b = jnp.broadcast_to(We1T.astype(bf)[None], (N, H, H))
        et3 = jax.lax.dot_general(W1b, r3, (((2,), (1,)), ((0,), (0,))),
                                  preferred_element_type=f32).astype(bf)
        bT = ((mm(We2T, hT) + be).astype(bf))
        e3 = jnp.maximum(et3 + bT[None, :, :], bf(0))
        logits = jnp.sum(wa.astype(bf)[None] * e3, axis=1).astype(f32) + ba
        msk = nei > 0
        lm = jnp.where(msk, logits, jnp.float32(-1e9))
        mx = jnp.max(lm, axis=1, keepdims=True)
        ex = jnp.exp(lm - mx)
        den = jnp.sum(ex, axis=1, keepdims=True)
        attn = jnp.where(msk, ex / den, 0.0).astype(bf)
        ctx3 = jnp.sum(attn[:, None, :] * e3, axis=2)
        return jnp.transpose(ctx3).astype(f32)

    enc_W = ld('enc_W')
    enc_b = ld('enc_b')
    l1_Wih = ld('l1_Wih')
    l1_Whh = ld('l1_Whh')
    l1_b = ld('l1_b')
    l2_Wih = ld('l2_Wih')
    l2_Whh = ld('l2_Whh')
    l2_b = ld('l2_b')
    dec_Wc = ld('dec_Wc')
    dec_Wo = ld('dec_Wo')
    dec_b = ld('dec_b')
    Wmu = ld('Wmu')
    bmu = ld('bmu')
    Wsc = ld('Wsc')
    bsc = ld('bsc')
    corr_W = ld('corr_W')
    corr_b = ld('corr_b')
    pl_p = (ld('pl_WrT'), ld('pl_br'), ld('pl_We1T'), ld('pl_We2T'),
            ld('pl_be'), ld('pl_wa'), ld('pl_ba'))
    plc_p = (ld('plc_WrT'), ld('plc_br'), ld('plc_We1T'), ld('plc_We2T'),
             ld('plc_be'), ld('plc_wa'), ld('plc_ba'))

    hT = ld('h0T')
    cT = ld('c0T')
    to = _OFF['trajT']
    for t in range(_OBS):
        xT = jnp.maximum(mm(enc_W, P_ref[to + 2 * t:to + 2 * t + 2, :])
                         + enc_b, 0.0)
        hT, cT = lstm(xT, hT, cT, l1_Wih, l1_Whh, l1_b)

    posT0 = ld('posT0')
    relgT = mm(ld('goal_W'), ld('goalT') - posT0) + ld('goal_b')

    def body(t, carry):
        outT, phT, pcT, posT, ctxT = carry
        xT = jnp.maximum(mm(dec_Wc, ctxT) + mm(dec_Wo, outT) + dec_b, 0.0)
        phT, pcT = lstm(xT, phT, pcT, l2_Wih, l2_Whh, l2_b)
        nei = mask_ref[pl.ds(t, 1)][0]
        ctx1 = pooling(posT, phT, nei, *pl_p)
        concT = ctx1 + phT + relgT
        muT = mm(Wmu, concT) + bmu
        scT = jnp.clip(mm(Wsc, concT) + bsc, -9.0, 4.0)
        pos_s = posT + muT
        ctx2 = pooling(pos_s, phT, nei, *plc_p)
        outP = mm(corr_W, ctx2) + corr_b + muT
        out_ref[pl.ds(t, 1)] = outP[None]
        out_ref[pl.ds(_PRED + t, 1)] = muT[None]
        out_ref[pl.ds(2 * _PRED + t, 1)] = scT[None]
        return (outP, phT, pcT, posT + outP, ctx1)

    init = (P_ref[to + 2 * (_OBS - 1):to + 2 * _OBS, :], hT,
            jnp.zeros_like(hT), posT0, jnp.zeros_like(hT))
    jax.lax.fori_loop(0, _PRED, body, init)


def kernel(traj_rel, obs_traj_pos, pred_traj_gt_pos, seq_start_end,
           nei_index, nei_num_index, sample_goal, params):
    p = params
    f32 = jnp.float32
    col = lambda v: v.reshape(-1, 1).astype(f32)

    def gate_reorder(W):
        return jnp.concatenate([W[:2 * _H], W[3 * _H:], W[2 * _H:3 * _H]], 0)

    vals = {
        'trajT': jnp.transpose(traj_rel[:_OBS], (0, 2, 1)).reshape(2 * _OBS, _N),
        'posT0': obs_traj_pos[-1].T,
        'goalT': sample_goal.T,
        'h0T': jnp.asarray(_H0T) if _H0T is not None else _hc_init()[0],
        'c0T': jnp.asarray(_C0T) if _C0T is not None else _hc_init()[1],
        'enc_W': p['enc_W'], 'enc_b': col(p['enc_b']),
        'goal_W': p['goal_W'], 'goal_b': col(p['goal_b']),
        'l1_Wih': gate_reorder(p['lstm1_Wih']),
        'l1_Whh': gate_reorder(p['lstm1_Whh']),
        'l1_b': gate_reorder(col(p['lstm1_bih'] + p['lstm1_bhh'])),
        'l2_Wih': gate_reorder(p['lstm2_Wih']),
        'l2_Whh': gate_reorder(p['lstm2_Whh']),
        'l2_b': gate_reorder(col(p['lstm2_bih'] + p['lstm2_bhh'])),
        'dec_Wc': p['dec_W'][:, :_H], 'dec_Wo': p['dec_W'][:, _H:],
        'dec_b': col(p['dec_b']),
        'Wmu': p['h2p_W'][:2], 'bmu': col(p['h2p_b'][:2]),
        'Wsc': p['h2p_W'][2:], 'bsc': col(p['h2p_b'][2:]),
        'corr_W': p['corr_W'], 'corr_b': col(p['corr_b']),
    }
    for pre in ('pl', 'plc'):
        vals[pre + '_WrT'] = p[pre + '_Wr'].T
        vals[pre + '_br'] = col(p[pre + '_br'])
        vals[pre + '_We1T'] = p[pre + '_We'][:_H].T
        vals[pre + '_We2T'] = p[pre + '_We'][_H:].T
        vals[pre + '_be'] = col(p[pre + '_be'])
        vals[pre + '_wa'] = p[pre + '_wa'].reshape(_H, 1)
        vals[pre + '_ba'] = p[pre + '_ba'].reshape(1, 1)

    blocks = []
    for nm, rr, cc in _SPEC:
        v = vals[nm].astype(f32)
        pr = -(-rr // 8) * 8
        blocks.append(jnp.pad(v, ((0, pr - rr), (0, _N - v.shape[1]))))
    P = jnp.concatenate(blocks, axis=0)

    out = pl.pallas_call(
        _trajgen_kernel,
        out_shape=jax.ShapeDtypeStruct((3 * _PRED, 2, _N), f32),
    )(P, nei_index)
    tr = lambda x: jnp.transpose(x, (0, 2, 1))
    return (tr(out[:_PRED]), tr(out[_PRED:2 * _PRED]), tr(out[2 * _PRED:]))


# speedup vs baseline: 1.4246x; 1.3006x over previous
import jax
import jax.numpy as jnp
import numpy as np
from jax.experimental import pallas as pl

_OBS = 8
_PRED = 12
_N = 256
_H = 16

def _hc_init():
    hk = jax.random.key(1)
    h = jax.random.normal(jax.random.fold_in(hk, 0), (_N, _H),
                          dtype=jnp.float32).T
    c = jax.random.normal(jax.random.fold_in(hk, 1), (_N, _H),
                          dtype=jnp.float32).T
    return h, c

try:
    _H0T, _C0T = (np.asarray(x).copy() for x in _hc_init())
except Exception:
    _H0T = _C0T = None

_SPEC = [
    ('trajT', 2 * _OBS, _N), ('posT0', 2, _N), ('goalT', 2, _N),
    ('h0T', _H, _N), ('c0T', _H, _N),
    ('enc_W', _H, 2), ('enc_b', _H, 1),
    ('goal_W', _H, 2), ('goal_b', _H, 1),
    ('l1_Wih', 4 * _H, _H), ('l1_Whh', 4 * _H, _H), ('l1_b', 4 * _H, 1),
    ('l2_Wih', 4 * _H, _H), ('l2_Whh', 4 * _H, _H), ('l2_b', 4 * _H, 1),
    ('dec_Wc', _H, _H), ('dec_Wo', _H, 2), ('dec_b', _H, 1),
    ('Wmu', 2, _H), ('bmu', 2, 1), ('Wsc', 2, _H), ('bsc', 2, 1),
    ('corr_W', 2, _H), ('corr_b', 2, 1),
    ('pl_WrT', _H, 2), ('pl_br', _H, 1),
    ('pl_We1T', _H, _H), ('pl_We2T', _H, _H),
    ('pl_be', _H, 1), ('pl_wa', _H, 1), ('pl_ba', 1, 1),
    ('plc_WrT', _H, 2), ('plc_br', _H, 1),
    ('plc_We1T', _H, _H), ('plc_We2T', _H, _H),
    ('plc_be', _H, 1), ('plc_wa', _H, 1), ('plc_ba', 1, 1),
]
_OFF = {}
_rows = 0
for _nm, _rr, _cc in _SPEC:
    _OFF[_nm] = _rows
    _rows += -(-_rr // 8) * 8
_P_ROWS = _rows


def _trajgen_kernel(P_ref, mask_ref, out_ref):
    f32 = jnp.float32
    H = _H
    N = _N

    def ld(nm):
        o = _OFF[nm]
        for s, rr, cc in _SPEC:
            if s == nm:
                return P_ref[o:o + rr, 0:cc]

    def mm(a, b):
        return jnp.dot(a, b, preferred_element_type=f32)

    def lstm(xT, hT, cT, Wih, Whh, b):
        g = mm(Wih, xT) + mm(Whh, hT) + b
        s = jax.nn.sigmoid(g[0:3 * H])
        gg = jnp.tanh(g[3 * H:4 * H])
        c2 = s[H:2 * H] * cT + s[0:H] * gg
        h2 = s[2 * H:3 * H] * jnp.tanh(c2)
        return h2, c2

    def pooling(posT, hT, nei, WrT, br, We1T, We2T, be, wa, ba):
        bf = jnp.bfloat16
        aT = mm(WrT, posT)
        a2t = (jax.lax.dot_general(posT, WrT, (((0,), (1,)), ((), ())),
                                   preferred_element_type=f32)
               + br[:, 0][None, :]).astype(bf)
        an = aT.astype(bf)
        r3 = jnp.maximum(a2t[:, :, None] - an[None, :, :], bf(0))
        W1b = jnp.broadcast_to(We1T.astype(bf)[None], (N, H, H))
        et3 = jax.lax.dot_general(W1b, r3, (((2,), (1,)), ((0,), (0,))),
                                  preferred_element_type=f32).astype(bf)
        bT = ((mm(We2T, hT) + be).astype(bf))
        e3 = jnp.maximum(et3 + bT[None, :, :], bf(0))
        wab = jnp.broadcast_to(wa.astype(bf).T[None], (N, 1, H))
        logits = jax.lax.dot_general(wab, e3, (((2,), (1,)), ((0,), (0,))),
                                     preferred_element_type=f32)[:, 0, :] + ba
        msk = nei > 0
        lm = jnp.where(msk, logits, jnp.float32(-1e9))
        mx = jnp.max(lm, axis=1, keepdims=True)
        ex = jnp.exp(lm - mx)
        den = jnp.sum(ex, axis=1, keepdims=True)
        attn = jnp.where(msk, ex / den, 0.0).astype(bf)
        ctx3 = jnp.sum(attn[:, None, :] * e3, axis=2)
        return jnp.transpose(ctx3).astype(f32)

    enc_W = ld('enc_W')
    enc_b = ld('enc_b')
    l1_Wih = ld('l1_Wih')
    l1_Whh = ld('l1_Whh')
    l1_b = ld('l1_b')
    l2_Wih = ld('l2_Wih')
    l2_Whh = ld('l2_Whh')
    l2_b = ld('l2_b')
    dec_Wc = ld('dec_Wc')
    dec_Wo = ld('dec_Wo')
    dec_b = ld('dec_b')
    Wmu = ld('Wmu')
    bmu = ld('bmu')
    Wsc = ld('Wsc')
    bsc = ld('bsc')
    corr_W = ld('corr_W')
    corr_b = ld('corr_b')
    pl_p = (ld('pl_WrT'), ld('pl_br'), ld('pl_We1T'), ld('pl_We2T'),
            ld('pl_be'), ld('pl_wa'), ld('pl_ba'))
    plc_p = (ld('plc_WrT'), ld('plc_br'), ld('plc_We1T'), ld('plc_We2T'),
             ld('plc_be'), ld('plc_wa'), ld('plc_ba'))

    hT = ld('h0T')
    cT = ld('c0T')
    to = _OFF['trajT']
    for t in range(_OBS):
        xT = jnp.maximum(mm(enc_W, P_ref[to + 2 * t:to + 2 * t + 2, :])
                         + enc_b, 0.0)
        hT, cT = lstm(xT, hT, cT, l1_Wih, l1_Whh, l1_b)

    posT0 = ld('posT0')
    relgT = mm(ld('goal_W'), ld('goalT') - posT0) + ld('goal_b')

    def body(t, carry):
        outT, phT, pcT, posT, ctxT = carry
        xT = jnp.maximum(mm(dec_Wc, ctxT) + mm(dec_Wo, outT) + dec_b, 0.0)
        phT, pcT = lstm(xT, phT, pcT, l2_Wih, l2_Whh, l2_b)
        nei = mask_ref[pl.ds(t, 1)][0]
        ctx1 = pooling(posT, phT, nei, *pl_p)
        concT = ctx1 + phT + relgT
        muT = mm(Wmu, concT) + bmu
        scT = jnp.clip(mm(Wsc, concT) + bsc, -9.0, 4.0)
        pos_s = posT + muT
        ctx2 = pooling(pos_s, phT, nei, *plc_p)
        outP = mm(corr_W, ctx2) + corr_b + muT
        out_ref[pl.ds(t, 1)] = outP[None]
        out_ref[pl.ds(_PRED + t, 1)] = muT[None]
        out_ref[pl.ds(2 * _PRED + t, 1)] = scT[None]
        return (outP, phT, pcT, posT + outP, ctx1)

    init = (P_ref[to + 2 * (_OBS - 1):to + 2 * _OBS, :], hT,
            jnp.zeros_like(hT), posT0, jnp.zeros_like(hT))
    jax.lax.fori_loop(0, _PRED, body, init)


def kernel(traj_rel, obs_traj_pos, pred_traj_gt_pos, seq_start_end,
           nei_index, nei_num_index, sample_goal, params):
    p = params
    f32 = jnp.float32
    col = lambda v: v.reshape(-1, 1).astype(f32)

    def gate_reorder(W):
        return jnp.concatenate([W[:2 * _H], W[3 * _H:], W[2 * _H:3 * _H]], 0)

    vals = {
        'trajT': jnp.transpose(traj_rel[:_OBS], (0, 2, 1)).reshape(2 * _OBS, _N),
        'posT0': obs_traj_pos[-1].T,
        'goalT': sample_goal.T,
        'h0T': jnp.asarray(_H0T) if _H0T is not None else _hc_init()[0],
        'c0T': jnp.asarray(_C0T) if _C0T is not None else _hc_init()[1],
        'enc_W': p['enc_W'], 'enc_b': col(p['enc_b']),
        'goal_W': p['goal_W'], 'goal_b': col(p['goal_b']),
        'l1_Wih': gate_reorder(p['lstm1_Wih']),
        'l1_Whh': gate_reorder(p['lstm1_Whh']),
        'l1_b': gate_reorder(col(p['lstm1_bih'] + p['lstm1_bhh'])),
        'l2_Wih': gate_reorder(p['lstm2_Wih']),
        'l2_Whh': gate_reorder(p['lstm2_Whh']),
        'l2_b': gate_reorder(col(p['lstm2_bih'] + p['lstm2_bhh'])),
        'dec_Wc': p['dec_W'][:, :_H], 'dec_Wo': p['dec_W'][:, _H:],
        'dec_b': col(p['dec_b']),
        'Wmu': p['h2p_W'][:2], 'bmu': col(p['h2p_b'][:2]),
        'Wsc': p['h2p_W'][2:], 'bsc': col(p['h2p_b'][2:]),
        'corr_W': p['corr_W'], 'corr_b': col(p['corr_b']),
    }
    for pre in ('pl', 'plc'):
        vals[pre + '_WrT'] = p[pre + '_Wr'].T
        vals[pre + '_br'] = col(p[pre + '_br'])
        vals[pre + '_We1T'] = p[pre + '_We'][:_H].T
        vals[pre + '_We2T'] = p[pre + '_We'][_H:].T
        vals[pre + '_be'] = col(p[pre + '_be'])
        vals[pre + '_wa'] = p[pre + '_wa'].reshape(_H, 1)
        vals[pre + '_ba'] = p[pre + '_ba'].reshape(1, 1)

    blocks = []
    for nm, rr, cc in _SPEC:
        v = vals[nm].astype(f32)
        pr = -(-rr // 8) * 8
        blocks.append(jnp.pad(v, ((0, pr - rr), (0, _N - v.shape[1]))))
    P = jnp.concatenate(blocks, axis=0)

    out = pl.pallas_call(
        _trajgen_kernel,
        out_shape=jax.ShapeDtypeStruct((3 * _PRED, 2, _N), f32),
    )(P, nei_index)
    tr = lambda x: jnp.transpose(x, (0, 2, 1))
    return (tr(out[:_PRED]), tr(out[_PRED:2 * _PRED]), tr(out[2 * _PRED:]))


# speedup vs baseline: 1.4940x; 1.0487x over previous
import jax
import jax.numpy as jnp
import numpy as np
from jax.experimental import pallas as pl

_OBS = 8
_PRED = 12
_N = 256
_H = 16

def _hc_init():
    hk = jax.random.key(1)
    h = jax.random.normal(jax.random.fold_in(hk, 0), (_N, _H),
                          dtype=jnp.float32).T
    c = jax.random.normal(jax.random.fold_in(hk, 1), (_N, _H),
                          dtype=jnp.float32).T
    return h, c

try:
    _H0T, _C0T = (np.asarray(x).copy() for x in _hc_init())
except Exception:
    _H0T = _C0T = None

_SPEC = [
    ('trajT', 2 * _OBS, _N), ('posT0', 2, _N), ('goalT', 2, _N),
    ('h0T', _H, _N), ('c0T', _H, _N),
    ('enc_W', _H, 2), ('enc_b', _H, 1),
    ('goal_W', _H, 2), ('goal_b', _H, 1),
    ('l1_Wih', 4 * _H, _H), ('l1_Whh', 4 * _H, _H), ('l1_b', 4 * _H, 1),
    ('l2_Wih', 4 * _H, _H), ('l2_Whh', 4 * _H, _H), ('l2_b', 4 * _H, 1),
    ('dec_Wc', _H, _H), ('dec_Wo', _H, 2), ('dec_b', _H, 1),
    ('Wmu', 2, _H), ('bmu', 2, 1), ('Wsc', 2, _H), ('bsc', 2, 1),
    ('corr_W', 2, _H), ('corr_b', 2, 1),
    ('pl_WrT', _H, 2), ('pl_br', _H, 1),
    ('pl_We1T', _H, _H), ('pl_We2T', _H, _H),
    ('pl_be', _H, 1), ('pl_wa', _H, 1), ('pl_ba', 1, 1),
    ('plc_WrT', _H, 2), ('plc_br', _H, 1),
    ('plc_We1T', _H, _H), ('plc_We2T', _H, _H),
    ('plc_be', _H, 1), ('plc_wa', _H, 1), ('plc_ba', 1, 1),
]
_OFF = {}
_rows = 0
for _nm, _rr, _cc in _SPEC:
    _OFF[_nm] = _rows
    _rows += -(-_rr // 8) * 8
_P_ROWS = _rows


def _trajgen_kernel(P_ref, mask_ref, out_ref):
    f32 = jnp.float32
    H = _H
    N = _N

    def ld(nm):
        o = _OFF[nm]
        for s, rr, cc in _SPEC:
            if s == nm:
                return P_ref[o:o + rr, 0:cc]

    def mm(a, b):
        return jnp.dot(a, b, preferred_element_type=f32)

    def lstm(xT, hT, cT, Wih, Whh, b):
        g = mm(Wih, xT) + mm(Whh, hT) + b
        s = jax.nn.sigmoid(g[0:3 * H])
        gg = jnp.tanh(g[3 * H:4 * H])
        c2 = s[H:2 * H] * cT + s[0:H] * gg
        h2 = s[2 * H:3 * H] * jnp.tanh(c2)
        return h2, c2

    def pooling(posT, hT, nei, WrT, br, We1T, We2T, be, wa, ba):
        bf = jnp.bfloat16
        aT = mm(WrT, posT)
        a2t = (jax.lax.dot_general(posT, WrT, (((0,), (1,)), ((), ())),
                                   preferred_element_type=f32)
               + br[:, 0][None, :]).astype(bf)
        an = aT.astype(bf)
        r3 = jnp.maximum(a2t[:, :, None] - an[None, :, :], bf(0))
        W1b = jnp.broadcast_to(We1T.astype(bf)[None], (N, H, H))
        et3 = jax.lax.dot_general(W1b, r3, (((2,), (1,)), ((0,), (0,))),
                                  preferred_element_type=f32).astype(bf)
        bT = ((mm(We2T, hT) + be).astype(bf))
        e3 = jnp.maximum(et3 + bT[None, :, :], bf(0))
        wab = jnp.broadcast_to(wa.astype(bf).T[None], (N, 1, H))
        logits = jax.lax.dot_general(wab, e3, (((2,), (1,)), ((0,), (0,))),
                                     preferred_element_type=f32)[:, 0, :] + ba
        msk = nei > 0
        lm = jnp.where(msk, logits, jnp.float32(-1e9))
        mx = jnp.max(lm, axis=1, keepdims=True)
        ex = jnp.exp(lm - mx)
        den = jnp.sum(ex, axis=1, keepdims=True)
        attn = jnp.where(msk, ex / den, 0.0).astype(bf)
        ctx3 = jax.lax.dot_general(attn[:, None, :], e3,
                                   (((2,), (2,)), ((0,), (0,))),
                                   preferred_element_type=f32)[:, 0, :]
        return jnp.transpose(ctx3)

    enc_W = ld('enc_W')
    enc_b = ld('enc_b')
    l1_Wih = ld('l1_Wih')
    l1_Whh = ld('l1_Whh')
    l1_b = ld('l1_b')
    l2_Wih = ld('l2_Wih')
    l2_Whh = ld('l2_Whh')
    l2_b = ld('l2_b')
    dec_Wc = ld('dec_Wc')
    dec_Wo = ld('dec_Wo')
    dec_b = ld('dec_b')
    Wmu = ld('Wmu')
    bmu = ld('bmu')
    Wsc = ld('Wsc')
    bsc = ld('bsc')
    corr_W = ld('corr_W')
    corr_b = ld('corr_b')
    pl_p = (ld('pl_WrT'), ld('pl_br'), ld('pl_We1T'), ld('pl_We2T'),
            ld('pl_be'), ld('pl_wa'), ld('pl_ba'))
    plc_p = (ld('plc_WrT'), ld('plc_br'), ld('plc_We1T'), ld('plc_We2T'),
             ld('plc_be'), ld('plc_wa'), ld('plc_ba'))

    hT = ld('h0T')
    cT = ld('c0T')
    to = _OFF['trajT']
    for t in range(_OBS):
        xT = jnp.maximum(mm(enc_W, P_ref[to + 2 * t:to + 2 * t + 2, :])
                         + enc_b, 0.0)
        hT, cT = lstm(xT, hT, cT, l1_Wih, l1_Whh, l1_b)

    posT0 = ld('posT0')
    relgT = mm(ld('goal_W'), ld('goalT') - posT0) + ld('goal_b')

    def body(t, carry):
        outT, phT, pcT, posT, ctxT = carry
        xT = jnp.maximum(mm(dec_Wc, ctxT) + mm(dec_Wo, outT) + dec_b, 0.0)
        phT, pcT = lstm(xT, phT, pcT, l2_Wih, l2_Whh, l2_b)
        nei = mask_ref[pl.ds(t, 1)][0]
        ctx1 = pooling(posT, phT, nei, *pl_p)
        concT = ctx1 + phT + relgT
        muT = mm(Wmu, concT) + bmu
        scT = jnp.clip(mm(Wsc, concT) + bsc, -9.0, 4.0)
        pos_s = posT + muT
        ctx2 = pooling(pos_s, phT, nei, *plc_p)
        outP = mm(corr_W, ctx2) + corr_b + muT
        out_ref[pl.ds(t, 1)] = outP[None]
        out_ref[pl.ds(_PRED + t, 1)] = muT[None]
        out_ref[pl.ds(2 * _PRED + t, 1)] = scT[None]
        return (outP, phT, pcT, posT + outP, ctx1)

    init = (P_ref[to + 2 * (_OBS - 1):to + 2 * _OBS, :], hT,
            jnp.zeros_like(hT), posT0, jnp.zeros_like(hT))
    jax.lax.fori_loop(0, _PRED, body, init)


def kernel(traj_rel, obs_traj_pos, pred_traj_gt_pos, seq_start_end,
           nei_index, nei_num_index, sample_goal, params):
    p = params
    f32 = jnp.float32
    col = lambda v: v.reshape(-1, 1).astype(f32)

    def gate_reorder(W):
        return jnp.concatenate([W[:2 * _H], W[3 * _H:], W[2 * _H:3 * _H]], 0)

    vals = {
        'trajT': jnp.transpose(traj_rel[:_OBS], (0, 2, 1)).reshape(2 * _OBS, _N),
        'posT0': obs_traj_pos[-1].T,
        'goalT': sample_goal.T,
        'h0T': jnp.asarray(_H0T) if _H0T is not None else _hc_init()[0],
        'c0T': jnp.asarray(_C0T) if _C0T is not None else _hc_init()[1],
        'enc_W': p['enc_W'], 'enc_b': col(p['enc_b']),
        'goal_W': p['goal_W'], 'goal_b': col(p['goal_b']),
        'l1_Wih': gate_reorder(p['lstm1_Wih']),
        'l1_Whh': gate_reorder(p['lstm1_Whh']),
        'l1_b': gate_reorder(col(p['lstm1_bih'] + p['lstm1_bhh'])),
        'l2_Wih': gate_reorder(p['lstm2_Wih']),
        'l2_Whh': gate_reorder(p['lstm2_Whh']),
        'l2_b': gate_reorder(col(p['lstm2_bih'] + p['lstm2_bhh'])),
        'dec_Wc': p['dec_W'][:, :_H], 'dec_Wo': p['dec_W'][:, _H:],
        'dec_b': col(p['dec_b']),
        'Wmu': p['h2p_W'][:2], 'bmu': col(p['h2p_b'][:2]),
        'Wsc': p['h2p_W'][2:], 'bsc': col(p['h2p_b'][2:]),
        'corr_W': p['corr_W'], 'corr_b': col(p['corr_b']),
    }
    for pre in ('pl', 'plc'):
        vals[pre + '_WrT'] = p[pre + '_Wr'].T
        vals[pre + '_br'] = col(p[pre + '_br'])
        vals[pre + '_We1T'] = p[pre + '_We'][:_H].T
        vals[pre + '_We2T'] = p[pre + '_We'][_H:].T
        vals[pre + '_be'] = col(p[pre + '_be'])
        vals[pre + '_wa'] = p[pre + '_wa'].reshape(_H, 1)
        vals[pre + '_ba'] = p[pre + '_ba'].reshape(1, 1)

    blocks = []
    for nm, rr, cc in _SPEC:
        v = vals[nm].astype(f32)
        pr = -(-rr // 8) * 8
        blocks.append(jnp.pad(v, ((0, pr - rr), (0, _N - v.shape[1]))))
    P = jnp.concatenate(blocks, axis=0)

    out = pl.pallas_call(
        _trajgen_kernel,
        out_shape=jax.ShapeDtypeStruct((3 * _PRED, 2, _N), f32),
    )(P, nei_index)
    tr = lambda x: jnp.transpose(x, (0, 2, 1))
    return (tr(out[:_PRED]), tr(out[_PRED:2 * _PRED]), tr(out[2 * _PRED:]))
